# baseline (speedup 1.0000x reference)
import jax
import jax.numpy as jnp
from jax.experimental import pallas as pl
from jax.experimental.pallas import tpu as pltpu


def _round_up(a, b):
    return (a + b - 1) // b * b


def _wprep_kernel(w_ref, g_ref, bias_ref, wa_ref, wb_ref, brow_ref):
    c_in, k, c_out = w_ref.shape
    w = w_ref[...]
    scale = g_ref[...] * jax.lax.rsqrt(jnp.sum(w * w, axis=(1, 2)))[:, None]
    ws = (w * scale[:, :, None]).astype(jnp.bfloat16)
    for p in range(4):
        wa_ref[0:c_in, p * c_out:(p + 1) * c_out] = ws[:, p + 4, :]
        wa_ref[c_in:2 * c_in, p * c_out:(p + 1) * c_out] = ws[:, p + 12, :]
        wb_ref[0:c_in, p * c_out:(p + 1) * c_out] = ws[:, p, :]
        wb_ref[c_in:2 * c_in, p * c_out:(p + 1) * c_out] = ws[:, p + 8, :]
    brow_ref[...] = jnp.concatenate([bias_ref[...]] * 4, axis=1)


def _convtr_kernel(xp_ref, wa_ref, wb_ref, b_ref, o_ref, nlc_ref):
    nb = xp_ref.shape[0]
    q = nlc_ref.shape[2] // 8
    c_out = o_ref.shape[1]
    bias = b_ref[...]
    n_half = c_out // 128
    c_in = xp_ref.shape[2]
    wa1, wa2 = wa_ref[0:c_in, :], wa_ref[c_in:2 * c_in, :]
    wb1, wb2 = wb_ref[0:c_in, :], wb_ref[c_in:2 * c_in, :]
    los, his = [], []
    for i in range(nb):
        x_q = xp_ref[i, 1:q + 1, :]
        los.append(jnp.dot(x_q, wa1, preferred_element_type=jnp.float32)
                   + jnp.dot(xp_ref[i, 0:q, :], wa2,
                             preferred_element_type=jnp.float32) + bias)
        his.append(jnp.dot(xp_ref[i, 2:q + 2, :], wb1,
                           preferred_element_type=jnp.float32)
                   + jnp.dot(x_q, wb2,
                             preferred_element_type=jnp.float32) + bias)
    for i in range(nb):
        for h in range(n_half):
            for p in range(4):
                c0 = p * c_out + h * 128
                nlc_ref[i, h, p:p + 8 * q:8, :] = los[i][:, c0:c0 + 128]
                nlc_ref[i, h, p + 4:p + 4 + 8 * q:8, :] = his[i][:, c0:c0 + 128]
            o_ref[i, h * 128:(h + 1) * 128, :] = jnp.transpose(nlc_ref[i, h], (1, 0))


def kernel(v, g, bias, x):
    c_in, c_out, k = v.shape
    n, _, l_in = x.shape
    s, pad = 8, 4
    l_out = (l_in - 1) * s - 2 * pad + k
    q_len = -(-l_out // s)

    wa, wb, bias_row = pl.pallas_call(
        _wprep_kernel,
        out_shape=(
            jax.ShapeDtypeStruct((2 * c_in, 4 * c_out), jnp.bfloat16),
            jax.ShapeDtypeStruct((2 * c_in, 4 * c_out), jnp.bfloat16),
            jax.ShapeDtypeStruct((1, 4 * c_out), jnp.float32),
        ),
    )(v.transpose(0, 2, 1), g.reshape(c_in, 1),
      bias.astype(jnp.float32)[None, :])

    l_pad = _round_up(q_len + 2, 8)
    xp = jnp.pad(x.transpose(0, 2, 1),
                 ((0, 0), (1, l_pad - l_in - 1), (0, 0))).astype(jnp.bfloat16)

    nb = 2 if n % 2 == 0 else 1
    out = pl.pallas_call(
        _convtr_kernel,
        out_shape=jax.ShapeDtypeStruct((n, c_out, q_len * s), jnp.float32),
        grid=(n // nb,),
        in_specs=[
            pl.BlockSpec((nb, l_pad, c_in), lambda b: (b, 0, 0)),
            pl.BlockSpec((2 * c_in, 4 * c_out), lambda b: (0, 0)),
            pl.BlockSpec((2 * c_in, 4 * c_out), lambda b: (0, 0)),
            pl.BlockSpec((1, 4 * c_out), lambda b: (0, 0)),
        ],
        out_specs=pl.BlockSpec((nb, c_out, q_len * s), lambda b: (b, 0, 0)),
        scratch_shapes=[pltpu.VMEM((nb, c_out // 128, q_len * s, 128), jnp.float32)],
        compiler_params=pltpu.CompilerParams(
            dimension_semantics=("parallel",)),
    )(xp, wa, wb, bias_row)

    return out[:, :, :l_out]


# speedup vs baseline: 1.0378x; 1.0378x over previous
import jax
import jax.numpy as jnp
from jax.experimental import pallas as pl
from jax.experimental.pallas import tpu as pltpu


def _round_up(a, b):
    return (a + b - 1) // b * b


def _wprep_kernel(w_ref, g_ref, bias_ref, wa_ref, wb_ref, brow_ref):
    c_in, k, c_out = w_ref.shape
    w = w_ref[...]
    scale = g_ref[...] * jax.lax.rsqrt(jnp.sum(w * w, axis=(1, 2)))[:, None]
    ws = (w * scale[:, :, None]).astype(jnp.bfloat16)
    for p in range(4):
        wa_ref[0:c_in, p * c_out:(p + 1) * c_out] = ws[:, p + 4, :]
        wa_ref[c_in:2 * c_in, p * c_out:(p + 1) * c_out] = ws[:, p + 12, :]
        wb_ref[0:c_in, p * c_out:(p + 1) * c_out] = ws[:, p, :]
        wb_ref[c_in:2 * c_in, p * c_out:(p + 1) * c_out] = ws[:, p + 8, :]
    brow_ref[...] = jnp.concatenate([bias_ref[...]] * 4, axis=1)


def _convtr_kernel(xp_ref, wa_ref, wb_ref, b_ref, o_ref, nlc_ref):
    nb = xp_ref.shape[0]
    q = nlc_ref.shape[2] // 8
    c_out = o_ref.shape[1]
    bias = b_ref[...]
    n_half = c_out // 128
    c_in = xp_ref.shape[2]
    wa1, wa2 = wa_ref[0:c_in, :], wa_ref[c_in:2 * c_in, :]
    wb1, wb2 = wb_ref[0:c_in, :], wb_ref[c_in:2 * c_in, :]
    los, his = [], []
    for i in range(nb):
        x_q = xp_ref[i, 1:q + 1, :]
        los.append(jnp.dot(x_q, wa1, preferred_element_type=jnp.float32)
                   + jnp.dot(xp_ref[i, 0:q, :], wa2,
                             preferred_element_type=jnp.float32) + bias)
        his.append(jnp.dot(xp_ref[i, 2:q + 2, :], wb1,
                           preferred_element_type=jnp.float32)
                   + jnp.dot(x_q, wb2,
                             preferred_element_type=jnp.float32) + bias)
    for i in range(nb):
        for h in range(n_half):
            for p in range(4):
                c0 = p * c_out + h * 128
                nlc_ref[i, h, p:p + 8 * q:8, :] = los[i][:, c0:c0 + 128]
                nlc_ref[i, h, p + 4:p + 4 + 8 * q:8, :] = his[i][:, c0:c0 + 128]
            o_ref[i, h * 128:(h + 1) * 128, :] = jnp.transpose(nlc_ref[i, h], (1, 0))


def kernel(v, g, bias, x):
    c_in, c_out, k = v.shape
    n, _, l_in = x.shape
    s, pad = 8, 4
    l_out = (l_in - 1) * s - 2 * pad + k
    q_len = -(-l_out // s)

    wa, wb, bias_row = pl.pallas_call(
        _wprep_kernel,
        out_shape=(
            jax.ShapeDtypeStruct((2 * c_in, 4 * c_out), jnp.bfloat16),
            jax.ShapeDtypeStruct((2 * c_in, 4 * c_out), jnp.bfloat16),
            jax.ShapeDtypeStruct((1, 4 * c_out), jnp.float32),
        ),
    )(v.transpose(0, 2, 1), g.reshape(c_in, 1),
      bias.astype(jnp.float32)[None, :])

    l_pad = _round_up(q_len + 2, 8)
    xp = jnp.pad(x.transpose(0, 2, 1),
                 ((0, 0), (1, l_pad - l_in - 1), (0, 0))).astype(jnp.bfloat16)

    nb = 4 if n % 4 == 0 else 1
    out = pl.pallas_call(
        _convtr_kernel,
        out_shape=jax.ShapeDtypeStruct((n, c_out, q_len * s), jnp.float32),
        grid=(n // nb,),
        in_specs=[
            pl.BlockSpec((nb, l_pad, c_in), lambda b: (b, 0, 0)),
            pl.BlockSpec((2 * c_in, 4 * c_out), lambda b: (0, 0)),
            pl.BlockSpec((2 * c_in, 4 * c_out), lambda b: (0, 0)),
            pl.BlockSpec((1, 4 * c_out), lambda b: (0, 0)),
        ],
        out_specs=pl.BlockSpec((nb, c_out, q_len * s), lambda b: (b, 0, 0)),
        scratch_shapes=[pltpu.VMEM((nb, c_out // 128, q_len * s, 128), jnp.float32)],
        compiler_params=pltpu.CompilerParams(
            dimension_semantics=("parallel",)),
    )(xp, wa, wb, bias_row)

    return out[:, :, :l_out]
